# Initial kernel scaffold; baseline (speedup 1.0000x reference)
import jax
import jax.numpy as jnp
from jax.experimental import pallas as pl


def kernel(anchor, positive):
    raise NotImplementedError("write your pallas kernel here")


# speedup vs baseline: 31.6577x; 31.6577x over previous
import jax
import jax.numpy as jnp
from jax.experimental import pallas as pl

_B = 4096
_D = 64
_BLK = 256
_K = 7
_MASK_VALUE = -10.0
_NEG_INF = -1e9


def _norm(x):
    n = jnp.sqrt(jnp.sum(x * x, axis=-1, keepdims=True))
    return x / jnp.maximum(n, 1e-12)


def _miner_body(a_ref, p_ref, out_ref):
    i = pl.program_id(0)
    a = a_ref[...]
    p = p_ref[...]
    sim = jax.lax.dot_general(
        a, p, (((1,), (1,)), ((), ())),
        preferred_element_type=jnp.float32)
    col = jax.lax.broadcasted_iota(jnp.int32, sim.shape, 1)
    row = jax.lax.broadcasted_iota(jnp.int32, sim.shape, 0)
    grow = row + i * _BLK
    pos = jnp.sum(jnp.where(col == grow, sim, 0.0), axis=1, keepdims=True)
    v = jnp.where(pos - sim > 0.0, sim, _MASK_VALUE)
    idxs = []
    for _ in range(_K):
        m = jnp.max(v, axis=1, keepdims=True)
        idx = jnp.min(jnp.where(v == m, col, _B), axis=1, keepdims=True)
        idxs.append(idx)
        v = jnp.where(col == idx, _NEG_INF, v)
    out_ref[...] = jnp.concatenate(idxs, axis=1)


def kernel(anchor, positive):
    anchor = _norm(anchor)
    positive = _norm(positive)
    grid = (_B // _BLK,)
    return pl.pallas_call(
        _miner_body,
        grid=grid,
        in_specs=[
            pl.BlockSpec((_BLK, _D), lambda i: (i, 0)),
            pl.BlockSpec((_B, _D), lambda i: (0, 0)),
        ],
        out_specs=pl.BlockSpec((_BLK, _K), lambda i: (i, 0)),
        out_shape=jax.ShapeDtypeStruct((_B, _K), jnp.int32),
    )(anchor, positive)


# speedup vs baseline: 38.3451x; 1.2112x over previous
import jax
import jax.numpy as jnp
from jax.experimental import pallas as pl

_B = 4096
_D = 64
_BLK = 256
_K = 7
_MASK_VALUE = -10.0
_NEG_INF = -1e9


def _norm(x):
    n = jnp.sqrt(jnp.sum(x * x, axis=-1, keepdims=True))
    return x / jnp.maximum(n, 1e-12)


def _miner_body(a_ref, p_ref, out_ref):
    i = pl.program_id(0)
    a = a_ref[...]
    p = p_ref[...]
    sim = jax.lax.dot_general(
        a, p, (((1,), (1,)), ((), ())),
        preferred_element_type=jnp.float32)
    col = jax.lax.broadcasted_iota(jnp.int32, sim.shape, 1).astype(jnp.float32)
    row = jax.lax.broadcasted_iota(jnp.int32, sim.shape, 0).astype(jnp.float32)
    grow = row + jnp.float32(i * _BLK)
    pos = jnp.sum(jnp.where(col == grow, sim, 0.0), axis=1, keepdims=True)
    v = jnp.where(pos - sim > 0.0, sim, _MASK_VALUE)
    idxs = []
    for _ in range(_K):
        m = jnp.max(v, axis=1, keepdims=True)
        idx = jnp.min(jnp.where(v == m, col, jnp.float32(_B)),
                      axis=1, keepdims=True)
        idxs.append(idx)
        v = jnp.where(col == idx, _NEG_INF, v)
    out_ref[...] = jnp.concatenate(idxs, axis=1).astype(jnp.int32)


def kernel(anchor, positive):
    anchor = _norm(anchor)
    positive = _norm(positive)
    grid = (_B // _BLK,)
    return pl.pallas_call(
        _miner_body,
        grid=grid,
        in_specs=[
            pl.BlockSpec((_BLK, _D), lambda i: (i, 0)),
            pl.BlockSpec((_B, _D), lambda i: (0, 0)),
        ],
        out_specs=pl.BlockSpec((_BLK, _K), lambda i: (i, 0)),
        out_shape=jax.ShapeDtypeStruct((_B, _K), jnp.int32),
    )(anchor, positive)


# speedup vs baseline: 38.8270x; 1.0126x over previous
import jax
import jax.numpy as jnp
from jax.experimental import pallas as pl

_B = 4096
_D = 64
_BLK = 512
_K = 7
_MASK_VALUE = -10.0
_NEG_INF = -1e9


def _norm(x):
    n = jnp.sqrt(jnp.sum(x * x, axis=-1, keepdims=True))
    return x / jnp.maximum(n, 1e-12)


def _miner_body(a_ref, p_ref, pb_ref, out_ref):
    a = a_ref[...]
    p = p_ref[...]
    pb = pb_ref[...]
    sim = jax.lax.dot_general(
        a, p, (((1,), (1,)), ((), ())),
        preferred_element_type=jnp.float32)
    col = jax.lax.broadcasted_iota(jnp.int32, sim.shape, 1).astype(jnp.float32)
    dblk = jax.lax.dot_general(
        a, pb, (((1,), (1,)), ((), ())),
        preferred_element_type=jnp.float32)
    dcol = jax.lax.broadcasted_iota(jnp.int32, (_BLK, _BLK), 1)
    drow = jax.lax.broadcasted_iota(jnp.int32, (_BLK, _BLK), 0)
    pos = jnp.sum(jnp.where(dcol == drow, dblk, 0.0), axis=1, keepdims=True)
    v = jnp.where(pos - sim > 0.0, sim, _MASK_VALUE)
    idxs = []
    for _ in range(_K):
        m = jnp.max(v, axis=1, keepdims=True)
        idx = jnp.min(jnp.where(v == m, col, jnp.float32(_B)),
                      axis=1, keepdims=True)
        idxs.append(idx)
        v = jnp.where(col == idx, _NEG_INF, v)
    out_ref[...] = jnp.concatenate(idxs, axis=1).astype(jnp.int32)


def kernel(anchor, positive):
    anchor = _norm(anchor)
    positive = _norm(positive)
    grid = (_B // _BLK,)
    return pl.pallas_call(
        _miner_body,
        grid=grid,
        in_specs=[
            pl.BlockSpec((_BLK, _D), lambda i: (i, 0)),
            pl.BlockSpec((_B, _D), lambda i: (0, 0)),
            pl.BlockSpec((_BLK, _D), lambda i: (i, 0)),
        ],
        out_specs=pl.BlockSpec((_BLK, _K), lambda i: (i, 0)),
        out_shape=jax.ShapeDtypeStruct((_B, _K), jnp.int32),
    )(anchor, positive, positive)


# speedup vs baseline: 39.3448x; 1.0133x over previous
import jax
import jax.numpy as jnp
from jax.experimental import pallas as pl

_B = 4096
_D = 64
_BLK = 1024
_K = 7
_MASK_VALUE = -10.0
_NEG_INF = -1e9


def _norm(x):
    n = jnp.sqrt(jnp.sum(x * x, axis=-1, keepdims=True))
    return x / jnp.maximum(n, 1e-12)


def _miner_body(a_ref, p_ref, pb_ref, out_ref):
    a = a_ref[...]
    p = p_ref[...]
    pb = pb_ref[...]
    sim = jax.lax.dot_general(
        a, p, (((1,), (1,)), ((), ())),
        preferred_element_type=jnp.float32)
    col = jax.lax.broadcasted_iota(jnp.int32, sim.shape, 1).astype(jnp.float32)
    dblk = jax.lax.dot_general(
        a, pb, (((1,), (1,)), ((), ())),
        preferred_element_type=jnp.float32)
    dcol = jax.lax.broadcasted_iota(jnp.int32, (_BLK, _BLK), 1)
    drow = jax.lax.broadcasted_iota(jnp.int32, (_BLK, _BLK), 0)
    pos = jnp.sum(jnp.where(dcol == drow, dblk, 0.0), axis=1, keepdims=True)
    v = jnp.where(pos - sim > 0.0, sim, _MASK_VALUE)
    idxs = []
    for _ in range(_K):
        m = jnp.max(v, axis=1, keepdims=True)
        idx = jnp.min(jnp.where(v == m, col, jnp.float32(_B)),
                      axis=1, keepdims=True)
        idxs.append(idx)
        v = jnp.where(col == idx, _NEG_INF, v)
    out_ref[...] = jnp.concatenate(idxs, axis=1).astype(jnp.int32)


def kernel(anchor, positive):
    anchor = _norm(anchor)
    positive = _norm(positive)
    grid = (_B // _BLK,)
    return pl.pallas_call(
        _miner_body,
        grid=grid,
        in_specs=[
            pl.BlockSpec((_BLK, _D), lambda i: (i, 0)),
            pl.BlockSpec((_B, _D), lambda i: (0, 0)),
            pl.BlockSpec((_BLK, _D), lambda i: (i, 0)),
        ],
        out_specs=pl.BlockSpec((_BLK, _K), lambda i: (i, 0)),
        out_shape=jax.ShapeDtypeStruct((_B, _K), jnp.int32),
    )(anchor, positive, positive)
